# baseline (speedup 1.0000x reference)
import functools

import jax
import jax.numpy as jnp
from jax import lax
from jax.experimental import pallas as pl
from jax.experimental.pallas import tpu as pltpu
from jax.experimental.pallas import tpu_sc as plsc

_B = 4
_S = 4096
_DIM = 6144
_NE = 80
_TOPK = 8
_HID = 256
_BALANCE_W = 0.3
_ENTROPY_W = 0.1
_Z_W = 0.0001

_SC_S = 1152
_TC_S = _S - _SC_S
_CHUNK = 128

_NC = 2
_NS = 16
_NW = _NC * _NS
_WPB = _NW // _B
_WROWS = _SC_S // _WPB
_R = 8
_NCHUNK = _WROWS // _R
_NSTRIP = _DIM // 16
_UNROLL = 8

_sc_mesh = plsc.VectorSubcoreMesh(core_axis_name="c", subcore_axis_name="s")


@functools.partial(
    pl.kernel, mesh=_sc_mesh,
    out_type=jax.ShapeDtypeStruct((_NW, _DIM), jnp.float32),
    scratch_types=[
        pltpu.VMEM((_R, _DIM), jnp.float32),
        pltpu.VMEM((_R, _DIM), jnp.float32),
        pltpu.VMEM((_DIM,), jnp.float32),
        pltpu.SemaphoreType.DMA,
        pltpu.SemaphoreType.DMA,
    ],
)
def _sc_partial_sum(x_hbm, out_hbm, buf0, buf1, acc, sem0, sem1):
    wid = lax.axis_index("s") * _NC + lax.axis_index("c")
    b = wid // _WPB
    row0 = _TC_S + (wid % _WPB) * _WROWS

    def zero_body(j, _):
        acc[pl.ds(16 * j, 16)] = jnp.zeros((16,), jnp.float32)
        return 0

    lax.fori_loop(0, _NSTRIP, zero_body, 0)

    def chunk_copy(q, buf, sem):
        src = x_hbm.at[b, pl.ds(row0 + q * _R, _R), :]
        return pltpu.make_async_copy(src, buf, sem)

    chunk_copy(0, buf0, sem0).start()
    chunk_copy(1, buf1, sem1).start()

    def reduce_buf(buf):
        def strip_body(jj, _):
            for u in range(_UNROLL):
                off = 16 * (_UNROLL * jj + u)
                vs = [buf[r, pl.ds(off, 16)] for r in range(_R)]
                while len(vs) > 1:
                    nxt = [vs[i] + vs[i + 1] for i in range(0, len(vs) - 1, 2)]
                    if len(vs) % 2:
                        nxt.append(vs[-1])
                    vs = nxt
                plsc.addupdate(acc.at[pl.ds(off, 16)], vs[0])
            return 0
        lax.fori_loop(0, _NSTRIP // _UNROLL, strip_body, 0)

    def g_body(g, _):
        chunk_copy(2 * g, buf0, sem0).wait()
        reduce_buf(buf0)

        @pl.when(2 * g + 2 < _NCHUNK)
        def _():
            chunk_copy(2 * g + 2, buf0, sem0).start()

        chunk_copy(2 * g + 1, buf1, sem1).wait()
        reduce_buf(buf1)

        @pl.when(2 * g + 3 < _NCHUNK)
        def _():
            chunk_copy(2 * g + 3, buf1, sem1).start()

        return 0

    lax.fori_loop(0, _NCHUNK // 2, g_body, 0)

    pltpu.sync_copy(acc, out_hbm.at[wid])


def _tc_sum_kernel(x_ref, o_ref):
    i = pl.program_id(1)
    partial = jnp.sum(jnp.reshape(x_ref[0], (_CHUNK // 8, 8, _DIM)), axis=0)

    @pl.when(i == 0)
    def _init():
        o_ref[0] = partial

    @pl.when(i != 0)
    def _acc():
        o_ref[0] += partial


def _tail_kernel(a_ref, b_ref, ws_ref, bs_ref, w1_ref, b1_ref, w2_ref,
                 b2_ref, temp_ref, w_out, i_out, u_out, l_out):
    sc_sum = jnp.sum(jnp.reshape(b_ref[...], (_B, _WPB, _DIM)), axis=1)
    tc_sum = jnp.sum(a_ref[...], axis=1)
    xm = (tc_sum + sc_sum) * (1.0 / _S)

    h = jax.lax.dot_general(xm, w1_ref[...], (((1,), (1,)), ((), ())),
                            preferred_element_type=jnp.float32)
    h = jnp.maximum(h + b1_ref[...], 0.0)
    pol = jax.lax.dot_general(h, w2_ref[...], (((1,), (1,)), ((), ())),
                              preferred_element_type=jnp.float32)
    pol = pol + b2_ref[...]
    pol = pol - jnp.max(pol, axis=-1, keepdims=True)
    pol = jnp.exp(pol)
    pol = pol / jnp.sum(pol, axis=-1, keepdims=True)

    t = jnp.maximum(temp_ref[0, 0], 0.1)
    base = jax.lax.dot_general(xm, ws_ref[...], (((1,), (1,)), ((), ())),
                               preferred_element_type=jnp.float32)
    base = (base + bs_ref[...]) / t

    sc = (base + pol) * 0.5
    sc = sc - jnp.max(sc, axis=-1, keepdims=True)
    sc = jnp.exp(sc)
    scores = sc / jnp.sum(sc, axis=-1, keepdims=True)

    entropy = -jnp.mean(jnp.sum(scores * jnp.log(scores + 1e-6), axis=-1))
    entropy_loss = -_ENTROPY_W * entropy
    bmax = jnp.max(base, axis=-1, keepdims=True)
    lse = jnp.log(jnp.sum(jnp.exp(base - bmax), axis=-1, keepdims=True)) + bmax
    z_loss = _Z_W * jnp.mean(lse * lse)

    lane = jax.lax.broadcasted_iota(jnp.int32, (_B, _NE), 1)
    rem = scores
    usage = jnp.zeros((1, _NE), jnp.float32)
    w_cols = []
    i_cols = []
    for _ in range(_TOPK):
        m = jnp.max(rem, axis=-1, keepdims=True)
        hit = rem == m
        idx = jnp.min(jnp.where(hit, lane, _NE), axis=-1, keepdims=True)
        w = m * t
        w_cols.append(w)
        i_cols.append(idx)
        usage = usage + jnp.sum(jnp.where(lane == idx, w, 0.0), axis=0,
                                keepdims=True)
        rem = jnp.where(lane == idx, -1.0, rem)

    u_out[...] = usage
    w_out[...] = jnp.concatenate(w_cols, axis=1)
    i_out[...] = jnp.concatenate(i_cols, axis=1)

    frac = usage / (jnp.mean(usage) + 1e-6)
    mu = jnp.mean(frac)
    var = jnp.sum((frac - mu) ** 2) / (_NE - 1)
    loss = _BALANCE_W * var + entropy_loss + z_loss
    l_out[...] = jnp.reshape(loss, (1, 1))


def kernel(x, Ws, bs, W1, b1, W2, b2, temp):
    bs2 = bs.reshape(1, _NE)
    b1_2 = b1.reshape(1, _HID)
    b2_2 = b2.reshape(1, _NE)
    temp2 = jnp.reshape(temp, (1, 1)).astype(jnp.float32)

    sc_planes = _sc_partial_sum(x)

    tc_part = pl.pallas_call(
        _tc_sum_kernel,
        grid=(_B, _TC_S // _CHUNK),
        in_specs=[pl.BlockSpec((1, _CHUNK, _DIM), lambda b, i: (b, i, 0))],
        out_specs=pl.BlockSpec((1, 8, _DIM), lambda b, i: (b, 0, 0)),
        out_shape=jax.ShapeDtypeStruct((_B, 8, _DIM), jnp.float32),
    )(x)

    weights, indices, usage, loss = pl.pallas_call(
        _tail_kernel,
        out_shape=[
            jax.ShapeDtypeStruct((_B, _TOPK), jnp.float32),
            jax.ShapeDtypeStruct((_B, _TOPK), jnp.int32),
            jax.ShapeDtypeStruct((1, _NE), jnp.float32),
            jax.ShapeDtypeStruct((1, 1), jnp.float32),
        ],
    )(tc_part, sc_planes, Ws, bs2, W1, b1_2, W2, b2_2, temp2)

    return (weights, indices, usage.reshape(_NE), loss[0, 0],
            jnp.asarray(0.0, jnp.float32), jnp.asarray(0.0, jnp.float32))


# speedup vs baseline: 1.0660x; 1.0660x over previous
import functools

import jax
import jax.numpy as jnp
from jax import lax
from jax.experimental import pallas as pl
from jax.experimental.pallas import tpu as pltpu
from jax.experimental.pallas import tpu_sc as plsc

_B = 4
_S = 4096
_DIM = 6144
_NE = 80
_TOPK = 8
_HID = 256
_BALANCE_W = 0.3
_ENTROPY_W = 0.1
_Z_W = 0.0001

_SC_S = 1024
_TC_S = _S - _SC_S
_CHUNK = 512

_NC = 2
_NS = 16
_NW = _NC * _NS
_WPB = _NW // _B
_WROWS = _SC_S // _WPB
_R = 8
_NCHUNK = _WROWS // _R
_NSTRIP = _DIM // 16
_UNROLL = 8

_sc_mesh = plsc.VectorSubcoreMesh(core_axis_name="c", subcore_axis_name="s")


@functools.partial(
    pl.kernel, mesh=_sc_mesh,
    out_type=jax.ShapeDtypeStruct((_NW, _DIM), jnp.float32),
    scratch_types=[
        pltpu.VMEM((_R, _DIM), jnp.float32),
        pltpu.VMEM((_R, _DIM), jnp.float32),
        pltpu.VMEM((_DIM,), jnp.float32),
        pltpu.SemaphoreType.DMA,
        pltpu.SemaphoreType.DMA,
    ],
)
def _sc_partial_sum(x_hbm, out_hbm, buf0, buf1, acc, sem0, sem1):
    wid = lax.axis_index("s") * _NC + lax.axis_index("c")
    b = wid // _WPB
    row0 = _TC_S + (wid % _WPB) * _WROWS

    def zero_body(j, _):
        acc[pl.ds(16 * j, 16)] = jnp.zeros((16,), jnp.float32)
        return 0

    lax.fori_loop(0, _NSTRIP, zero_body, 0)

    def chunk_copy(q, buf, sem):
        src = x_hbm.at[b, pl.ds(row0 + q * _R, _R), :]
        return pltpu.make_async_copy(src, buf, sem)

    chunk_copy(0, buf0, sem0).start()
    chunk_copy(1, buf1, sem1).start()

    def reduce_buf(buf):
        def strip_body(jj, _):
            for u in range(_UNROLL):
                off = 16 * (_UNROLL * jj + u)
                vs = [buf[r, pl.ds(off, 16)] for r in range(_R)]
                while len(vs) > 1:
                    nxt = [vs[i] + vs[i + 1] for i in range(0, len(vs) - 1, 2)]
                    if len(vs) % 2:
                        nxt.append(vs[-1])
                    vs = nxt
                plsc.addupdate(acc.at[pl.ds(off, 16)], vs[0])
            return 0
        lax.fori_loop(0, _NSTRIP // _UNROLL, strip_body, 0)

    def g_body(g, _):
        chunk_copy(2 * g, buf0, sem0).wait()
        reduce_buf(buf0)

        @pl.when(2 * g + 2 < _NCHUNK)
        def _():
            chunk_copy(2 * g + 2, buf0, sem0).start()

        chunk_copy(2 * g + 1, buf1, sem1).wait()
        reduce_buf(buf1)

        @pl.when(2 * g + 3 < _NCHUNK)
        def _():
            chunk_copy(2 * g + 3, buf1, sem1).start()

        return 0

    lax.fori_loop(0, _NCHUNK // 2, g_body, 0)

    pltpu.sync_copy(acc, out_hbm.at[wid])


def _tc_sum_kernel(x_ref, o_ref):
    i = pl.program_id(1)
    partial = jnp.sum(jnp.reshape(x_ref[0], (_CHUNK // 8, 8, _DIM)), axis=0)

    @pl.when(i == 0)
    def _init():
        o_ref[0] = partial

    @pl.when(i != 0)
    def _acc():
        o_ref[0] += partial


def _tail_kernel(a_ref, b_ref, ws_ref, bs_ref, w1_ref, b1_ref, w2_ref,
                 b2_ref, temp_ref, w_out, i_out, u_out, l_out):
    sc_sum = jnp.sum(jnp.reshape(b_ref[...], (_B, _WPB, _DIM)), axis=1)
    tc_sum = jnp.sum(a_ref[...], axis=1)
    xm = (tc_sum + sc_sum) * (1.0 / _S)

    h = jax.lax.dot_general(xm, w1_ref[...], (((1,), (1,)), ((), ())),
                            preferred_element_type=jnp.float32)
    h = jnp.maximum(h + b1_ref[...], 0.0)
    pol = jax.lax.dot_general(h, w2_ref[...], (((1,), (1,)), ((), ())),
                              preferred_element_type=jnp.float32)
    pol = pol + b2_ref[...]
    pol = pol - jnp.max(pol, axis=-1, keepdims=True)
    pol = jnp.exp(pol)
    pol = pol / jnp.sum(pol, axis=-1, keepdims=True)

    t = jnp.maximum(temp_ref[0, 0], 0.1)
    base = jax.lax.dot_general(xm, ws_ref[...], (((1,), (1,)), ((), ())),
                               preferred_element_type=jnp.float32)
    base = (base + bs_ref[...]) / t

    sc = (base + pol) * 0.5
    sc = sc - jnp.max(sc, axis=-1, keepdims=True)
    sc = jnp.exp(sc)
    scores = sc / jnp.sum(sc, axis=-1, keepdims=True)

    entropy = -jnp.mean(jnp.sum(scores * jnp.log(scores + 1e-6), axis=-1))
    entropy_loss = -_ENTROPY_W * entropy
    bmax = jnp.max(base, axis=-1, keepdims=True)
    lse = jnp.log(jnp.sum(jnp.exp(base - bmax), axis=-1, keepdims=True)) + bmax
    z_loss = _Z_W * jnp.mean(lse * lse)

    lane = jax.lax.broadcasted_iota(jnp.int32, (_B, _NE), 1)
    rem = scores
    usage = jnp.zeros((1, _NE), jnp.float32)
    w_cols = []
    i_cols = []
    for _ in range(_TOPK):
        m = jnp.max(rem, axis=-1, keepdims=True)
        hit = rem == m
        idx = jnp.min(jnp.where(hit, lane, _NE), axis=-1, keepdims=True)
        w = m * t
        w_cols.append(w)
        i_cols.append(idx)
        usage = usage + jnp.sum(jnp.where(lane == idx, w, 0.0), axis=0,
                                keepdims=True)
        rem = jnp.where(lane == idx, -1.0, rem)

    u_out[...] = usage
    w_out[...] = jnp.concatenate(w_cols, axis=1)
    i_out[...] = jnp.concatenate(i_cols, axis=1)

    frac = usage / (jnp.mean(usage) + 1e-6)
    mu = jnp.mean(frac)
    var = jnp.sum((frac - mu) ** 2) / (_NE - 1)
    loss = _BALANCE_W * var + entropy_loss + z_loss
    l_out[...] = jnp.reshape(loss, (1, 1))


def kernel(x, Ws, bs, W1, b1, W2, b2, temp):
    bs2 = bs.reshape(1, _NE)
    b1_2 = b1.reshape(1, _HID)
    b2_2 = b2.reshape(1, _NE)
    temp2 = jnp.reshape(temp, (1, 1)).astype(jnp.float32)

    sc_planes = _sc_partial_sum(x)

    tc_part = pl.pallas_call(
        _tc_sum_kernel,
        grid=(_B, _TC_S // _CHUNK),
        in_specs=[pl.BlockSpec((1, _CHUNK, _DIM), lambda b, i: (b, i, 0))],
        out_specs=pl.BlockSpec((1, 8, _DIM), lambda b, i: (b, 0, 0)),
        out_shape=jax.ShapeDtypeStruct((_B, 8, _DIM), jnp.float32),
    )(x)

    weights, indices, usage, loss = pl.pallas_call(
        _tail_kernel,
        out_shape=[
            jax.ShapeDtypeStruct((_B, _TOPK), jnp.float32),
            jax.ShapeDtypeStruct((_B, _TOPK), jnp.int32),
            jax.ShapeDtypeStruct((1, _NE), jnp.float32),
            jax.ShapeDtypeStruct((1, 1), jnp.float32),
        ],
    )(tc_part, sc_planes, Ws, bs2, W1, b1_2, W2, b2_2, temp2)

    return (weights, indices, usage.reshape(_NE), loss[0, 0],
            jnp.asarray(0.0, jnp.float32), jnp.asarray(0.0, jnp.float32))
